# Initial kernel scaffold; baseline (speedup 1.0000x reference)
import jax
import jax.numpy as jnp
from jax.experimental import pallas as pl


def kernel(x, u):
    raise NotImplementedError("write your pallas kernel here")


# speedup vs baseline: 2.8962x; 2.8962x over previous
import functools

import jax
import jax.numpy as jnp
from jax import lax
from jax.experimental import pallas as pl
from jax.experimental.pallas import tpu as pltpu
from jax.experimental.pallas import tpu_sc as plsc

P = 32
M = 512
N = 512
IMG = M * N
ROWS_PER_CHUNK = 8
C = ROWS_PER_CHUNK * N
NUM_CHUNKS = IMG // C
VECS = C // 16


def _floor_parts(v):
    ti = v.astype(jnp.int32)
    tf = ti.astype(jnp.float32)
    fi = jnp.where(tf > v, ti - 1, ti)
    w = v - fi.astype(jnp.float32)
    return fi, w


def _warp_body(x_hbm, dx_hbm, dy_hbm, out_hbm,
               img_s, dxv, dyv,
               i00, i01, i10, i11,
               w00, w01, w10, w11,
               v00, v01, v10, v11,
               outv, sem_in, sem_g, sem_out):
    nc = 2
    cid = lax.axis_index("c")
    sid = lax.axis_index("s")
    wid = sid * nc + cid

    @pl.when(sid == 0)
    def _stage():
        pltpu.sync_copy(x_hbm, img_s)

    plsc.subcore_barrier()

    pbase = (wid // 8) * IMG
    ubase = wid * IMG

    lane = lax.broadcasted_iota(jnp.int32, (16,), 0).astype(jnp.float32)

    def chunk_body(ck, _):
        base = ubase + ck * C
        cpx = pltpu.async_copy(dx_hbm.at[pl.ds(base, C)], dxv, sem_in)
        cpy = pltpu.async_copy(dy_hbm.at[pl.ds(base, C)], dyv, sem_in)
        cpx.wait()
        cpy.wait()

        r0 = ck * ROWS_PER_CHUNK

        def vec_body(t, _):
            off = t * 16
            iy = (r0 + off // N).astype(jnp.float32)
            jx = (off % N).astype(jnp.float32)

            dxl = dxv[pl.ds(off, 16)]
            dyl = dyv[pl.ds(off, 16)]
            xs = lane + jx + dxl
            ys = iy + dyl
            xs = jnp.minimum(jnp.maximum(xs, -2.0), 513.0)
            ys = jnp.minimum(jnp.maximum(ys, -2.0), 513.0)

            x0, wx = _floor_parts(xs)
            y0, wy = _floor_parts(ys)

            vx0 = (x0 >= 0) & (x0 <= N - 1)
            vx1 = (x0 >= -1) & (x0 <= N - 2)
            vy0 = (y0 >= 0) & (y0 <= M - 1)
            vy1 = (y0 >= -1) & (y0 <= M - 2)

            x0c = jnp.minimum(jnp.maximum(x0, 0), N - 1)
            x1c = jnp.minimum(jnp.maximum(x0 + 1, 0), N - 1)
            y0c = jnp.minimum(jnp.maximum(y0, 0), M - 1)
            y1c = jnp.minimum(jnp.maximum(y0 + 1, 0), M - 1)

            yb0 = y0c * N + pbase
            yb1 = y1c * N + pbase
            i00[pl.ds(off, 16)] = yb0 + x0c
            i01[pl.ds(off, 16)] = yb0 + x1c
            i10[pl.ds(off, 16)] = yb1 + x0c
            i11[pl.ds(off, 16)] = yb1 + x1c

            ox = 1.0 - wx
            oy = 1.0 - wy
            zero = jnp.zeros((16,), jnp.float32)
            w00[pl.ds(off, 16)] = jnp.where(vx0 & vy0, ox * oy, zero)
            w01[pl.ds(off, 16)] = jnp.where(vx1 & vy0, wx * oy, zero)
            w10[pl.ds(off, 16)] = jnp.where(vx0 & vy1, ox * wy, zero)
            w11[pl.ds(off, 16)] = jnp.where(vx1 & vy1, wx * wy, zero)
            return _

        lax.fori_loop(0, VECS, vec_body, None)

        g0 = pltpu.async_copy(img_s.at[i00], v00, sem_g)
        g1 = pltpu.async_copy(img_s.at[i01], v01, sem_g)
        g2 = pltpu.async_copy(img_s.at[i10], v10, sem_g)
        g3 = pltpu.async_copy(img_s.at[i11], v11, sem_g)
        g0.wait()
        g1.wait()
        g2.wait()
        g3.wait()

        def mix_body(t, _):
            off = t * 16
            acc = (w00[pl.ds(off, 16)] * v00[pl.ds(off, 16)]
                   + w01[pl.ds(off, 16)] * v01[pl.ds(off, 16)]
                   + w10[pl.ds(off, 16)] * v10[pl.ds(off, 16)]
                   + w11[pl.ds(off, 16)] * v11[pl.ds(off, 16)])
            outv[pl.ds(off, 16)] = acc
            return _

        lax.fori_loop(0, VECS, mix_body, None)

        pltpu.sync_copy(outv, out_hbm.at[pl.ds(base, C)])
        return _

    lax.fori_loop(0, NUM_CHUNKS, chunk_body, None)


@jax.jit
def _warp_call(x_flat, dx_flat, dy_flat):
    mesh = plsc.VectorSubcoreMesh(core_axis_name="c", subcore_axis_name="s")
    f = pl.kernel(
        _warp_body,
        out_type=jax.ShapeDtypeStruct((P * IMG,), jnp.float32),
        mesh=mesh,
        scratch_types=[
            pltpu.VMEM_SHARED((4 * IMG,), jnp.float32),
            pltpu.VMEM((C,), jnp.float32),
            pltpu.VMEM((C,), jnp.float32),
            pltpu.VMEM((C,), jnp.int32),
            pltpu.VMEM((C,), jnp.int32),
            pltpu.VMEM((C,), jnp.int32),
            pltpu.VMEM((C,), jnp.int32),
            pltpu.VMEM((C,), jnp.float32),
            pltpu.VMEM((C,), jnp.float32),
            pltpu.VMEM((C,), jnp.float32),
            pltpu.VMEM((C,), jnp.float32),
            pltpu.VMEM((C,), jnp.float32),
            pltpu.VMEM((C,), jnp.float32),
            pltpu.VMEM((C,), jnp.float32),
            pltpu.VMEM((C,), jnp.float32),
            pltpu.VMEM((C,), jnp.float32),
            pltpu.SemaphoreType.DMA,
            pltpu.SemaphoreType.DMA,
            pltpu.SemaphoreType.DMA,
        ],
    )
    return f(x_flat, dx_flat, dy_flat)


def kernel(x, u):
    x_flat = x.reshape(-1)
    dx_flat = u[..., 0].reshape(-1)
    dy_flat = u[..., 1].reshape(-1)
    out = _warp_call(x_flat, dx_flat, dy_flat)
    return out.reshape(u.shape[:-1])
